# Initial kernel scaffold; baseline (speedup 1.0000x reference)
import jax
import jax.numpy as jnp
from jax.experimental import pallas as pl


def kernel(x, node_embs, node_gamma, node_beta, running_mean, running_var, Wt, P):
    raise NotImplementedError("write your pallas kernel here")


# speedup vs baseline: 1.4165x; 1.4165x over previous
import functools

import jax
import jax.numpy as jnp
from jax.experimental import pallas as pl
from jax.experimental.pallas import tpu as pltpu

N_NODES = 12
FANOUT = 4
EPSILON = 0.01
EPS_BN = 1e-5
C = 192
EMB = 64
B = 128
H = 14
W = 14
HW = H * W
BB = 8
NBLK = B // BB


def _routing_update(i, pooled, ad, cfT, P_v, embsT_v, rm, rv, g, b):
    tlo, thi = i + 1, min(i + FANOUT, N_NODES - 1)
    q = jnp.dot(pooled, P_v, preferred_element_type=jnp.float32)
    sc = jnp.dot(q, embsT_v, preferred_element_type=jnp.float32)
    sc = (sc - rm) / jnp.sqrt(rv + EPS_BN) * g + b
    lane = jax.lax.broadcasted_iota(jnp.int32, (B, N_NODES), 1)
    win = (lane >= tlo) & (lane <= thi)
    scm = jnp.where(win, sc, -1e30)
    mx = jnp.max(scm, axis=1, keepdims=True)
    e = jnp.where(win, jnp.exp(scm - mx), 0.0)
    trans = e / jnp.sum(e, axis=1, keepdims=True)
    if i == 0:
        a = jnp.ones((B, 1), jnp.float32)
    else:
        a = ad[:, i:i + 1]
    sent = a * trans
    ad1 = jnp.where(lane == i, 0.0, ad) + sent
    mk = (ad1 > EPSILON).astype(jnp.float32)
    s = 1.0 / (jnp.sum(ad1 * mk, axis=1, keepdims=True) + 1e-12)
    mrow = mk * s
    ad2 = ad1 * mrow
    psub = jax.lax.broadcasted_iota(jnp.int32, (B, N_NODES, N_NODES), 1)
    cf1 = cfT + jnp.where(psub == i, sent[:, None, :], 0.0)
    cf2 = cf1 * mrow[:, None, :]
    return ad2, cf2


def _step0_body(x_ref, P_ref, embsT_ref, rm_ref, rv_ref, g_ref, b_ref,
                ad_out, cf_out, psum):
    pid = pl.program_id(0)

    @pl.when(pid < NBLK)
    def _():
        psum[pl.ds(pid * BB, BB), :] = jnp.sum(x_ref[...], axis=1)

    @pl.when(pid == NBLK)
    def _():
        pooled = psum[...] * (1.0 / HW)
        ad0 = jnp.zeros((B, N_NODES), jnp.float32)
        cf0 = jnp.zeros((B, N_NODES, N_NODES), jnp.float32)
        ad2, cf2 = _routing_update(0, pooled, ad0, cf0, P_ref[...],
                                   embsT_ref[...], rm_ref[...], rv_ref[...],
                                   g_ref[...], b_ref[...])
        ad_out[...] = ad2
        cf_out[...] = cf2


def _stepi_body(i, npred, *refs):
    (c_ref, *t_refs), rest = refs[:1 + npred], refs[1 + npred:]
    (wtT_ref, ad_ref, cf_ref, P_ref, embsT_ref, rm_ref, rv_ref, g_ref,
     b_ref, feat_ref, ad_out, cf_out, psum) = rest
    pid = pl.program_id(0)

    @pl.when(pid < NBLK)
    def _():
        for s in range(BB):
            gb = pid * BB + s
            agg = c_ref[gb, 0] * t_refs[0][s]
            for o in range(1, npred):
                agg = agg + c_ref[gb, o] * t_refs[o][s]
            y = jnp.dot(agg, wtT_ref[...], preferred_element_type=jnp.float32)
            y = jnp.maximum(y, 0.0)
            feat_ref[s] = y
            psum[pl.ds(gb, 1), :] = jnp.sum(y, axis=0, keepdims=True)

    @pl.when(pid == NBLK)
    def _():
        pooled = psum[...] * (1.0 / HW)
        ad2, cf2 = _routing_update(i, pooled, ad_ref[...], cf_ref[...],
                                   P_ref[...], embsT_ref[...], rm_ref[...],
                                   rv_ref[...], g_ref[...], b_ref[...])
        ad_out[...] = ad2
        cf_out[...] = cf2


def _combine_body(c_ref, t0, t1, t2, t3, out_ref):
    pid = pl.program_id(0)
    ts = (t0, t1, t2, t3)
    for s in range(BB):
        gb = pid * BB + s
        acc = c_ref[gb, 0] * ts[0][s]
        for o in range(1, 4):
            acc = acc + c_ref[gb, o] * ts[o][s]
        out_ref[s] = acc


def _feat_spec():
    return pl.BlockSpec((BB, HW, C), lambda pid: (jnp.minimum(pid, NBLK - 1), 0, 0))


def _const_spec(shape):
    nd = len(shape)
    return pl.BlockSpec(shape, lambda pid: (0,) * nd)


def kernel(x, node_embs, node_gamma, node_beta, running_mean, running_var, Wt, P):
    f32 = jnp.float32
    xr = jnp.transpose(x, (0, 2, 3, 1)).reshape(B, HW, C)
    WtT = jnp.transpose(Wt, (0, 2, 1))
    embsT = node_embs.T
    rm = running_mean.reshape(1, N_NODES)
    rv = running_var.reshape(1, N_NODES)
    g = node_gamma.reshape(1, N_NODES)
    b = node_beta.reshape(1, N_NODES)

    small_specs = [_const_spec((C, EMB)), _const_spec((EMB, N_NODES))] + \
                  [_const_spec((1, N_NODES))] * 4
    state_out = (jax.ShapeDtypeStruct((B, N_NODES), f32),
                 jax.ShapeDtypeStruct((B, N_NODES, N_NODES), f32))
    state_out_specs = [_const_spec((B, N_NODES)),
                       _const_spec((B, N_NODES, N_NODES))]

    ad, cf = pl.pallas_call(
        _step0_body,
        grid=(NBLK + 1,),
        in_specs=[_feat_spec()] + small_specs,
        out_specs=state_out_specs,
        out_shape=state_out,
        scratch_shapes=[pltpu.VMEM((B, C), f32)],
    )(xr, P, embsT, rm, rv, g, b)

    tfeat = [xr]
    for i in range(1, N_NODES - 1):
        p0 = max(0, i - FANOUT)
        npred = i - p0
        cnext = cf[:, p0:i, i]
        feat, ad, cf = pl.pallas_call(
            functools.partial(_stepi_body, i, npred),
            grid=(NBLK + 1,),
            in_specs=[pl.BlockSpec(memory_space=pltpu.SMEM)] +
                     [_feat_spec()] * npred +
                     [_const_spec((C, C)),
                      _const_spec((B, N_NODES)),
                      _const_spec((B, N_NODES, N_NODES))] + small_specs,
            out_specs=[_feat_spec()] + state_out_specs,
            out_shape=(jax.ShapeDtypeStruct((B, HW, C), f32),) + state_out,
            scratch_shapes=[pltpu.VMEM((B, C), f32)],
        )(cnext, *[tfeat[p] for p in range(p0, i)], WtT[i], ad, cf,
          P, embsT, rm, rv, g, b)
        tfeat.append(feat)

    c_last = cf[:, N_NODES - 1 - FANOUT:N_NODES - 1, N_NODES - 1]
    out2d = pl.pallas_call(
        _combine_body,
        grid=(NBLK,),
        in_specs=[pl.BlockSpec(memory_space=pltpu.SMEM)] +
                 [pl.BlockSpec((BB, HW, C), lambda pid: (pid, 0, 0))] * 4,
        out_specs=pl.BlockSpec((BB, HW, C), lambda pid: (pid, 0, 0)),
        out_shape=jax.ShapeDtypeStruct((B, HW, C), f32),
    )(c_last, tfeat[7], tfeat[8], tfeat[9], tfeat[10])

    return out2d.reshape(B, H, W, C).transpose(0, 3, 1, 2)


# speedup vs baseline: 1.8082x; 1.2765x over previous
import functools

import jax
import jax.numpy as jnp
from jax.experimental import pallas as pl
from jax.experimental.pallas import tpu as pltpu

N_NODES = 12
FANOUT = 4
EPSILON = 0.01
EPS_BN = 1e-5
C = 192
EMB = 64
B = 128
H = 14
W = 14
HW = H * W
BB = 8
NBLK = B // BB


def _routing_update(i, pooled, ad, cfT, P_v, embsT_v, rm, rv, g, b):
    tlo, thi = i + 1, min(i + FANOUT, N_NODES - 1)
    q = jnp.dot(pooled, P_v, preferred_element_type=jnp.float32)
    sc = jnp.dot(q, embsT_v, preferred_element_type=jnp.float32)
    sc = (sc - rm) / jnp.sqrt(rv + EPS_BN) * g + b
    lane = jax.lax.broadcasted_iota(jnp.int32, (B, N_NODES), 1)
    win = (lane >= tlo) & (lane <= thi)
    scm = jnp.where(win, sc, -1e30)
    mx = jnp.max(scm, axis=1, keepdims=True)
    e = jnp.where(win, jnp.exp(scm - mx), 0.0)
    trans = e / jnp.sum(e, axis=1, keepdims=True)
    if i == 0:
        a = jnp.ones((B, 1), jnp.float32)
    else:
        a = ad[:, i:i + 1]
    sent = a * trans
    ad1 = jnp.where(lane == i, 0.0, ad) + sent
    mk = (ad1 > EPSILON).astype(jnp.float32)
    s = 1.0 / (jnp.sum(ad1 * mk, axis=1, keepdims=True) + 1e-12)
    mrow = mk * s
    ad2 = ad1 * mrow
    psub = jax.lax.broadcasted_iota(jnp.int32, (B, N_NODES, N_NODES), 1)
    cf1 = cfT + jnp.where(psub == i, sent[:, None, :], 0.0)
    cf2 = cf1 * mrow[:, None, :]
    return ad2, cf2


def _step0_body(x_ref, P_ref, embsT_ref, rm_ref, rv_ref, g_ref, b_ref,
                ad_out, cf_out, psum):
    pid = pl.program_id(0)

    @pl.when(pid < NBLK)
    def _():
        psum[pl.ds(pid * BB, BB), :] = jnp.sum(x_ref[...], axis=1)

    @pl.when(pid == NBLK)
    def _():
        pooled = psum[...] * (1.0 / HW)
        ad0 = jnp.zeros((B, N_NODES), jnp.float32)
        cf0 = jnp.zeros((B, N_NODES, N_NODES), jnp.float32)
        ad2, cf2 = _routing_update(0, pooled, ad0, cf0, P_ref[...],
                                   embsT_ref[...], rm_ref[...], rv_ref[...],
                                   g_ref[...], b_ref[...])
        ad_out[...] = ad2
        cf_out[...] = cf2


def _stepi_body(i, npred, *refs):
    (c_ref, *t_refs), rest = refs[:1 + npred], refs[1 + npred:]
    (wtT_ref, ad_ref, cf_ref, P_ref, embsT_ref, rm_ref, rv_ref, g_ref,
     b_ref, feat_ref, ad_out, cf_out, psum) = rest
    pid = pl.program_id(0)

    @pl.when(pid < NBLK)
    def _():
        for s in range(BB):
            gb = pid * BB + s
            agg = c_ref[gb, 0] * t_refs[0][s].astype(jnp.float32)
            for o in range(1, npred):
                agg = agg + c_ref[gb, o] * t_refs[o][s].astype(jnp.float32)
            y = jnp.dot(agg, wtT_ref[...], preferred_element_type=jnp.float32)
            y = jnp.maximum(y, 0.0)
            feat_ref[s] = y.astype(jnp.bfloat16)
            psum[pl.ds(gb, 1), :] = jnp.sum(y, axis=0, keepdims=True)

    @pl.when(pid == NBLK)
    def _():
        pooled = psum[...] * (1.0 / HW)
        ad2, cf2 = _routing_update(i, pooled, ad_ref[...], cf_ref[...],
                                   P_ref[...], embsT_ref[...], rm_ref[...],
                                   rv_ref[...], g_ref[...], b_ref[...])
        ad_out[...] = ad2
        cf_out[...] = cf2


def _combine_body(c_ref, t0, t1, t2, t3, out_ref):
    pid = pl.program_id(0)
    ts = (t0, t1, t2, t3)
    for s in range(BB):
        gb = pid * BB + s
        acc = c_ref[gb, 0] * ts[0][s].astype(jnp.float32)
        for o in range(1, 4):
            acc = acc + c_ref[gb, o] * ts[o][s].astype(jnp.float32)
        out_ref[s] = acc


def _feat_spec():
    return pl.BlockSpec((BB, HW, C), lambda pid: (jnp.minimum(pid, NBLK - 1), 0, 0))


def _const_spec(shape):
    nd = len(shape)
    return pl.BlockSpec(shape, lambda pid: (0,) * nd)


def kernel(x, node_embs, node_gamma, node_beta, running_mean, running_var, Wt, P):
    f32 = jnp.float32
    xr = jnp.transpose(x, (0, 2, 3, 1)).reshape(B, HW, C)
    WtT = jnp.transpose(Wt, (0, 2, 1))
    embsT = node_embs.T
    rm = running_mean.reshape(1, N_NODES)
    rv = running_var.reshape(1, N_NODES)
    g = node_gamma.reshape(1, N_NODES)
    b = node_beta.reshape(1, N_NODES)

    small_specs = [_const_spec((C, EMB)), _const_spec((EMB, N_NODES))] + \
                  [_const_spec((1, N_NODES))] * 4
    state_out = (jax.ShapeDtypeStruct((B, N_NODES), f32),
                 jax.ShapeDtypeStruct((B, N_NODES, N_NODES), f32))
    state_out_specs = [_const_spec((B, N_NODES)),
                       _const_spec((B, N_NODES, N_NODES))]

    ad, cf = pl.pallas_call(
        _step0_body,
        grid=(NBLK + 1,),
        in_specs=[_feat_spec()] + small_specs,
        out_specs=state_out_specs,
        out_shape=state_out,
        scratch_shapes=[pltpu.VMEM((B, C), f32)],
    )(xr, P, embsT, rm, rv, g, b)

    tfeat = [xr.astype(jnp.bfloat16)]
    for i in range(1, N_NODES - 1):
        p0 = max(0, i - FANOUT)
        npred = i - p0
        cnext = cf[:, p0:i, i]
        feat, ad, cf = pl.pallas_call(
            functools.partial(_stepi_body, i, npred),
            grid=(NBLK + 1,),
            in_specs=[pl.BlockSpec(memory_space=pltpu.SMEM)] +
                     [_feat_spec()] * npred +
                     [_const_spec((C, C)),
                      _const_spec((B, N_NODES)),
                      _const_spec((B, N_NODES, N_NODES))] + small_specs,
            out_specs=[_feat_spec()] + state_out_specs,
            out_shape=(jax.ShapeDtypeStruct((B, HW, C), jnp.bfloat16),)
                      + state_out,
            scratch_shapes=[pltpu.VMEM((B, C), f32)],
        )(cnext, *[tfeat[p] for p in range(p0, i)], WtT[i], ad, cf,
          P, embsT, rm, rv, g, b)
        tfeat.append(feat)

    c_last = cf[:, N_NODES - 1 - FANOUT:N_NODES - 1, N_NODES - 1]
    out2d = pl.pallas_call(
        _combine_body,
        grid=(NBLK,),
        in_specs=[pl.BlockSpec(memory_space=pltpu.SMEM)] +
                 [pl.BlockSpec((BB, HW, C), lambda pid: (pid, 0, 0))] * 4,
        out_specs=pl.BlockSpec((BB, HW, C), lambda pid: (pid, 0, 0)),
        out_shape=jax.ShapeDtypeStruct((B, HW, C), f32),
    )(c_last, tfeat[7], tfeat[8], tfeat[9], tfeat[10])

    return out2d.reshape(B, H, W, C).transpose(0, 3, 1, 2)


# speedup vs baseline: 1.8084x; 1.0001x over previous
import functools

import jax
import jax.numpy as jnp
from jax.experimental import pallas as pl
from jax.experimental.pallas import tpu as pltpu

N_NODES = 12
FANOUT = 4
EPSILON = 0.01
EPS_BN = 1e-5
C = 192
EMB = 64
B = 128
H = 14
W = 14
HW = H * W
BB = 8
NBLK = B // BB


def _routing_update(i, pooled, ad, cfT, P_v, embsT_v, rm, rv, g, b):
    tlo, thi = i + 1, min(i + FANOUT, N_NODES - 1)
    q = jnp.dot(pooled, P_v, preferred_element_type=jnp.float32)
    sc = jnp.dot(q, embsT_v, preferred_element_type=jnp.float32)
    sc = (sc - rm) / jnp.sqrt(rv + EPS_BN) * g + b
    lane = jax.lax.broadcasted_iota(jnp.int32, (B, N_NODES), 1)
    win = (lane >= tlo) & (lane <= thi)
    scm = jnp.where(win, sc, -1e30)
    mx = jnp.max(scm, axis=1, keepdims=True)
    e = jnp.where(win, jnp.exp(scm - mx), 0.0)
    trans = e / jnp.sum(e, axis=1, keepdims=True)
    if i == 0:
        a = jnp.ones((B, 1), jnp.float32)
    else:
        a = ad[:, i:i + 1]
    sent = a * trans
    ad1 = jnp.where(lane == i, 0.0, ad) + sent
    mk = (ad1 > EPSILON).astype(jnp.float32)
    s = 1.0 / (jnp.sum(ad1 * mk, axis=1, keepdims=True) + 1e-12)
    mrow = mk * s
    ad2 = ad1 * mrow
    psub = jax.lax.broadcasted_iota(jnp.int32, (B, N_NODES, N_NODES), 1)
    cf1 = cfT + jnp.where(psub == i, sent[:, None, :], 0.0)
    cf2 = cf1 * mrow[:, None, :]
    return ad2, cf2


def _step0_body(x_ref, P_ref, embsT_ref, rm_ref, rv_ref, g_ref, b_ref,
                ad_out, cf_out, psum):
    pid = pl.program_id(0)

    @pl.when(pid < NBLK)
    def _():
        psum[pl.ds(pid * BB, BB), :] = jnp.sum(x_ref[...], axis=1)

    @pl.when(pid == NBLK)
    def _():
        pooled = psum[...] * (1.0 / HW)
        ad0 = jnp.zeros((B, N_NODES), jnp.float32)
        cf0 = jnp.zeros((B, N_NODES, N_NODES), jnp.float32)
        ad2, cf2 = _routing_update(0, pooled, ad0, cf0, P_ref[...],
                                   embsT_ref[...], rm_ref[...], rv_ref[...],
                                   g_ref[...], b_ref[...])
        ad_out[...] = ad2
        cf_out[...] = cf2


def _stepi_body(i, npred, *refs):
    (c_ref, *t_refs), rest = refs[:1 + npred], refs[1 + npred:]
    (wtT_ref, ad_ref, cf_ref, P_ref, embsT_ref, rm_ref, rv_ref, g_ref,
     b_ref, feat_ref, ad_out, cf_out, psum) = rest
    pid = pl.program_id(0)

    @pl.when(pid < NBLK)
    def _():
        for s in range(BB):
            gb = pid * BB + s
            agg = c_ref[gb, 0] * t_refs[0][s].astype(jnp.float32)
            for o in range(1, npred):
                agg = agg + c_ref[gb, o] * t_refs[o][s].astype(jnp.float32)
            y = jnp.dot(agg.astype(jnp.bfloat16), wtT_ref[...],
                        preferred_element_type=jnp.float32)
            y = jnp.maximum(y, 0.0)
            feat_ref[s] = y.astype(jnp.bfloat16)
            psum[pl.ds(gb, 1), :] = jnp.sum(y, axis=0, keepdims=True)

    @pl.when(pid == NBLK)
    def _():
        pooled = psum[...] * (1.0 / HW)
        ad2, cf2 = _routing_update(i, pooled, ad_ref[...], cf_ref[...],
                                   P_ref[...], embsT_ref[...], rm_ref[...],
                                   rv_ref[...], g_ref[...], b_ref[...])
        ad_out[...] = ad2
        cf_out[...] = cf2


def _combine_body(c_ref, t0, t1, t2, t3, out_ref):
    pid = pl.program_id(0)
    ts = (t0, t1, t2, t3)
    for s in range(BB):
        gb = pid * BB + s
        acc = c_ref[gb, 0] * ts[0][s].astype(jnp.float32)
        for o in range(1, 4):
            acc = acc + c_ref[gb, o] * ts[o][s].astype(jnp.float32)
        out_ref[s] = acc


def _feat_spec():
    return pl.BlockSpec((BB, HW, C), lambda pid: (jnp.minimum(pid, NBLK - 1), 0, 0))


def _const_spec(shape):
    nd = len(shape)
    return pl.BlockSpec(shape, lambda pid: (0,) * nd)


def kernel(x, node_embs, node_gamma, node_beta, running_mean, running_var, Wt, P):
    f32 = jnp.float32
    xr = jnp.transpose(x, (0, 2, 3, 1)).reshape(B, HW, C)
    WtT = jnp.transpose(Wt, (0, 2, 1)).astype(jnp.bfloat16)
    embsT = node_embs.T
    rm = running_mean.reshape(1, N_NODES)
    rv = running_var.reshape(1, N_NODES)
    g = node_gamma.reshape(1, N_NODES)
    b = node_beta.reshape(1, N_NODES)

    small_specs = [_const_spec((C, EMB)), _const_spec((EMB, N_NODES))] + \
                  [_const_spec((1, N_NODES))] * 4
    state_out = (jax.ShapeDtypeStruct((B, N_NODES), f32),
                 jax.ShapeDtypeStruct((B, N_NODES, N_NODES), f32))
    state_out_specs = [_const_spec((B, N_NODES)),
                       _const_spec((B, N_NODES, N_NODES))]

    ad, cf = pl.pallas_call(
        _step0_body,
        grid=(NBLK + 1,),
        in_specs=[_feat_spec()] + small_specs,
        out_specs=state_out_specs,
        out_shape=state_out,
        scratch_shapes=[pltpu.VMEM((B, C), f32)],
    )(xr, P, embsT, rm, rv, g, b)

    tfeat = [xr.astype(jnp.bfloat16)]
    for i in range(1, N_NODES - 1):
        p0 = max(0, i - FANOUT)
        npred = i - p0
        cnext = cf[:, p0:i, i]
        feat, ad, cf = pl.pallas_call(
            functools.partial(_stepi_body, i, npred),
            grid=(NBLK + 1,),
            in_specs=[pl.BlockSpec(memory_space=pltpu.SMEM)] +
                     [_feat_spec()] * npred +
                     [_const_spec((C, C)),
                      _const_spec((B, N_NODES)),
                      _const_spec((B, N_NODES, N_NODES))] + small_specs,
            out_specs=[_feat_spec()] + state_out_specs,
            out_shape=(jax.ShapeDtypeStruct((B, HW, C), jnp.bfloat16),)
                      + state_out,
            scratch_shapes=[pltpu.VMEM((B, C), f32)],
        )(cnext, *[tfeat[p] for p in range(p0, i)], WtT[i], ad, cf,
          P, embsT, rm, rv, g, b)
        tfeat.append(feat)

    c_last = cf[:, N_NODES - 1 - FANOUT:N_NODES - 1, N_NODES - 1]
    out2d = pl.pallas_call(
        _combine_body,
        grid=(NBLK,),
        in_specs=[pl.BlockSpec(memory_space=pltpu.SMEM)] +
                 [pl.BlockSpec((BB, HW, C), lambda pid: (pid, 0, 0))] * 4,
        out_specs=pl.BlockSpec((BB, HW, C), lambda pid: (pid, 0, 0)),
        out_shape=jax.ShapeDtypeStruct((B, HW, C), f32),
    )(c_last, tfeat[7], tfeat[8], tfeat[9], tfeat[10])

    return out2d.reshape(B, H, W, C).transpose(0, 3, 1, 2)


# speedup vs baseline: 2.7843x; 1.5396x over previous
import jax
import jax.numpy as jnp
from jax.experimental import pallas as pl
from jax.experimental.pallas import tpu as pltpu

N_NODES = 12
FANOUT = 4
EPSILON = 0.01
EPS_BN = 1e-5
C = 192
EMB = 64
B = 128
H = 14
W = 14
HW = H * W
BB = 16
NBLK = B // BB
NSTEP = N_NODES - 1


def _mega_body(x_ref, wtT_ref, P_ref, embsT_ref, rm_ref, rv_ref, g_ref,
               b_ref, out_ref, ring, ad_s, cf_s, cn_s, psum, stage, dsem):
    i = pl.program_id(0)
    sub = pl.program_id(1)
    gb = jnp.minimum(sub, NBLK - 1) * BB

    @pl.when((i == 0) & (sub < NBLK))
    def _():
        cp = pltpu.make_async_copy(x_ref.at[:, pl.ds(gb, BB), :], stage, dsem)
        cp.start()
        cp.wait()
        xb = stage[...]
        ring[0, :, pl.ds(gb, BB), :] = xb.astype(jnp.bfloat16)
        z = jnp.zeros((HW, BB, C), jnp.bfloat16)
        ring[1, :, pl.ds(gb, BB), :] = z
        ring[2, :, pl.ds(gb, BB), :] = z
        ring[3, :, pl.ds(gb, BB), :] = z
        psum[pl.ds(gb, BB), :] = jnp.sum(xb, axis=0)

    @pl.when((i >= 1) & (i <= NSTEP - 1) & (sub < NBLK))
    def _():
        agg = jnp.zeros((HW, BB, C), jnp.float32)
        for o in range(1, FANOUT + 1):
            pc = jnp.maximum(i - o, 0)
            slot = jax.lax.rem(pc, 4)
            c = cn_s[pl.ds(pc, 1), pl.ds(gb, BB), :]
            c = jnp.where(i - o >= 0, c, 0.0)
            t = ring[slot, :, pl.ds(gb, BB), :].astype(jnp.float32)
            agg = agg + c * t
        a2 = agg.reshape(HW * BB, C).astype(jnp.bfloat16)
        y = jnp.dot(a2, wtT_ref[i - 1], preferred_element_type=jnp.float32)
        y = jnp.maximum(y, 0.0).reshape(HW, BB, C)
        ring[jax.lax.rem(i, 4), :, pl.ds(gb, BB), :] = y.astype(jnp.bfloat16)
        psum[pl.ds(gb, BB), :] = jnp.sum(y, axis=0)

    @pl.when((i <= NSTEP - 1) & (sub == NBLK))
    def _():
        pooled = psum[...] * (1.0 / HW)
        q = jnp.dot(pooled, P_ref[...], preferred_element_type=jnp.float32)
        sc = jnp.dot(q, embsT_ref[...], preferred_element_type=jnp.float32)
        sc = (sc - rm_ref[...]) / jnp.sqrt(rv_ref[...] + EPS_BN) \
            * g_ref[...] + b_ref[...]
        lane = jax.lax.broadcasted_iota(jnp.int32, (B, N_NODES), 1)
        win = (lane >= i + 1) & (lane <= jnp.minimum(i + FANOUT, N_NODES - 1))
        scm = jnp.where(win, sc, -1e30)
        mx = jnp.max(scm, axis=1, keepdims=True)
        e = jnp.where(win, jnp.exp(scm - mx), 0.0)
        trans = e / jnp.sum(e, axis=1, keepdims=True)
        ad = jnp.where(i == 0, 0.0, ad_s[...])
        a = jnp.where(i == 0, 1.0,
                      jnp.sum(jnp.where(lane == i, ad, 0.0), axis=1,
                              keepdims=True))
        sent = a * trans
        ad1 = jnp.where(lane == i, 0.0, ad) + sent
        mk = (ad1 > EPSILON).astype(jnp.float32)
        s = 1.0 / (jnp.sum(ad1 * mk, axis=1, keepdims=True) + 1e-12)
        mrow = mk * s
        ad_s[...] = ad1 * mrow
        cf = jnp.where(i == 0, 0.0, cf_s[...])
        psub = jax.lax.broadcasted_iota(jnp.int32, (N_NODES, B, N_NODES), 0)
        cf = cf + jnp.where(psub == i, sent[None, :, :], 0.0)
        cf = cf * mrow[None, :, :]
        cf_s[...] = cf
        lane3 = jax.lax.broadcasted_iota(jnp.int32, (N_NODES, B, N_NODES), 2)
        cn_s[...] = jnp.sum(jnp.where(lane3 == i + 1, cf, 0.0), axis=2,
                            keepdims=True)

    @pl.when((i == NSTEP) & (sub < NBLK))
    def _():
        acc = jnp.zeros((HW, BB, C), jnp.float32)
        for p in range(N_NODES - 1 - FANOUT, N_NODES - 1):
            c = cn_s[p, pl.ds(gb, BB), :]
            t = ring[p % 4, :, pl.ds(gb, BB), :].astype(jnp.float32)
            acc = acc + c[None] * t
        stage[...] = acc
        cp = pltpu.make_async_copy(stage, out_ref.at[:, pl.ds(gb, BB), :],
                                   dsem)
        cp.start()
        cp.wait()


def kernel(x, node_embs, node_gamma, node_beta, running_mean, running_var,
           Wt, P):
    f32 = jnp.float32
    xr = jnp.transpose(x, (2, 3, 0, 1)).reshape(HW, B, C)
    WtT = jnp.transpose(Wt[1:N_NODES - 1], (0, 2, 1)).astype(jnp.bfloat16)
    embsT = node_embs.T
    rm = running_mean.reshape(1, N_NODES)
    rv = running_var.reshape(1, N_NODES)
    g = node_gamma.reshape(1, N_NODES)
    b = node_beta.reshape(1, N_NODES)

    def c0(shape):
        nd = len(shape)
        return pl.BlockSpec(shape, lambda i, s: (0,) * nd)

    out2d = pl.pallas_call(
        _mega_body,
        grid=(NSTEP + 1, NBLK + 1),
        in_specs=[
            pl.BlockSpec(memory_space=pl.ANY),
            c0((N_NODES - 2, C, C)),
            c0((C, EMB)), c0((EMB, N_NODES)),
            c0((1, N_NODES)), c0((1, N_NODES)),
            c0((1, N_NODES)), c0((1, N_NODES)),
        ],
        out_specs=pl.BlockSpec(memory_space=pl.ANY),
        out_shape=jax.ShapeDtypeStruct((HW, B, C), f32),
        scratch_shapes=[
            pltpu.VMEM((4, HW, B, C), jnp.bfloat16),
            pltpu.VMEM((B, N_NODES), f32),
            pltpu.VMEM((N_NODES, B, N_NODES), f32),
            pltpu.VMEM((N_NODES, B, 1), f32),
            pltpu.VMEM((B, C), f32),
            pltpu.VMEM((HW, BB, C), f32),
            pltpu.SemaphoreType.DMA,
        ],
        compiler_params=pltpu.CompilerParams(
            vmem_limit_bytes=64 * 1024 * 1024),
    )(xr, WtT, P, embsT, rm, rv, g, b)

    return out2d.reshape(H, W, B, C).transpose(2, 3, 0, 1)
